# baseline (speedup 1.0000x reference)
import functools

import jax
import jax.numpy as jnp
from jax import lax
from jax.experimental import pallas as pl
from jax.experimental.pallas import tpu as pltpu
from jax.experimental.pallas import tpu_sc as plsc

B, T_IN, N, C_IN = 32, 40, 50, 1
HID, OUT, K, E = 32, 64, 3, 800
NG = B * (T_IN - 2)
T1 = T_IN - 2
T2 = T_IN - 4
T3 = T_IN - 6
NP = 56
APAD = NP * NP
GPW = NG // 32
F = N * OUT
TROW = T2 * NP


def _sc_body(ei_hbm, out_hbm, ei_a, ei_b, slot_a, slot_b, sem_in, sem_oa,
             sem_ob):
    cid = lax.axis_index("c")
    sid = lax.axis_index("s")
    wid = sid * 2 + cid
    g0 = wid * GPW
    zf = jnp.zeros((16,), jnp.float32)
    ones = jnp.full((16,), 1.0, jnp.float32)
    pltpu.async_copy(ei_hbm.at[g0], ei_a, sem_in)

    def one(g, ei_v, ei_nxt, slot_v, sem_out, first):
        pltpu.make_async_copy(ei_hbm.at[g], ei_v, sem_in).wait()

        @pl.when(g + 1 < g0 + GPW)
        def _():
            pltpu.async_copy(ei_hbm.at[g + 1], ei_nxt, sem_in)

        @pl.when(jnp.logical_not(first))
        def _():
            pltpu.make_async_copy(slot_v, out_hbm.at[g], sem_out).wait()
        for i in range(APAD // 16):
            slot_v[pl.ds(i * 16, 16)] = zf
        for i in range(E // 16):
            r = ei_v[0, pl.ds(i * 16, 16)]
            c = ei_v[1, pl.ds(i * 16, 16)]
            flat = r * NP + c
            ew = jnp.where(r != c, ones, zf)
            plsc.addupdate_scatter(slot_v, [flat], ew)
        pltpu.async_copy(slot_v, out_hbm.at[g], sem_out)

    def body(j, carry):
        g = g0 + 2 * j
        one(g, ei_a, ei_b, slot_a, sem_oa, j == 0)
        one(g + 1, ei_b, ei_a, slot_b, sem_ob, j == 0)
        return carry

    lax.fori_loop(0, GPW // 2, body, 0)
    pltpu.make_async_copy(slot_a, out_hbm.at[g0], sem_oa).wait()
    pltpu.make_async_copy(slot_b, out_hbm.at[g0], sem_ob).wait()


@functools.lru_cache(maxsize=1)
def _get_sc_kernel():
    mesh = plsc.VectorSubcoreMesh(core_axis_name="c", subcore_axis_name="s")
    return pl.kernel(
        _sc_body,
        mesh=mesh,
        compiler_params=pltpu.CompilerParams(needs_layout_passes=False),
        out_type=jax.ShapeDtypeStruct((NG, APAD), jnp.float32),
        scratch_types=[
            pltpu.VMEM((2, E), jnp.int32),
            pltpu.VMEM((2, E), jnp.int32),
            pltpu.VMEM((APAD,), jnp.float32),
            pltpu.VMEM((APAD,), jnp.float32),
            pltpu.SemaphoreType.DMA,
            pltpu.SemaphoreType.DMA,
            pltpu.SemaphoreType.DMA,
        ],
    )


def _tc1_body(x_ref, w_ref, b_ref, o_ref):
    x = x_ref[0]
    a = [x[k:k + T1][:, :, None] for k in range(3)]

    def conv(j):
        acc = b_ref[j][None, None, :]
        for k in range(3):
            acc = acc + a[k] * w_ref[j, k][None, None, :]
        return acc

    P = conv(0)
    Q = conv(1)
    R = conv(2)
    H = P * (1.0 / (1.0 + jnp.exp(-Q))) + R
    o_ref[0] = jnp.maximum(H, 0.0)


def _cheb_tc2_body(a_ref, x_ref, w1_ref0, b1_ref0, cw_ref, cb_ref, w0_ref,
                   w1_ref, w2_ref, b_ref, st_ref, v_ref, s_ref):
    x = x_ref[0]
    xs = [x[k:k + T1][:, :, None] for k in range(3)]

    def conv1(j):
        acc = b1_ref0[j][None, None, :]
        for k in range(3):
            acc = acc + xs[k] * w1_ref0[j, k][None, None, :]
        return acc

    P1 = conv1(0)
    Q1 = conv1(1)
    R1 = conv1(2)
    H1 = jnp.maximum(P1 * (1.0 / (1.0 + jnp.exp(-Q1))) + R1, 0.0)

    W0 = cw_ref[0]
    W1 = cw_ref[1]
    W2 = cw_ref[2]
    cb = cb_ref[...]
    X_all = H1.reshape(T1 * NP, HID)
    A_all = a_ref[0].reshape(T1 * NP, NP)
    deg = jnp.sum(A_all, axis=1, keepdims=True)
    safe = jnp.where(deg > 0, deg, 1.0)
    dis = jnp.where(deg > 0, lax.rsqrt(safe), 0.0)
    ndis = -dis
    dx = dis * X_all
    t1 = [jnp.dot(a_ref[0, i], dx[i * NP:(i + 1) * NP],
                  preferred_element_type=jnp.float32) for i in range(T1)]
    Tx1 = ndis * jnp.concatenate(t1, axis=0)
    dt = dis * Tx1
    t2 = [jnp.dot(a_ref[0, i], dt[i * NP:(i + 1) * NP],
                  preferred_element_type=jnp.float32) for i in range(T1)]
    Tx2 = 2.0 * (ndis * jnp.concatenate(t2, axis=0)) - X_all
    Tg = (jnp.dot(X_all, W0, preferred_element_type=jnp.float32)
          + jnp.dot(Tx1, W1, preferred_element_type=jnp.float32)
          + jnp.dot(Tx2, W2, preferred_element_type=jnp.float32) + cb)
    Tb = jnp.maximum(Tg, 0.0)

    bb = b_ref[...]
    ST = st_ref[...]
    Y = (jnp.dot(Tb[0:TROW], w0_ref[...],
                 preferred_element_type=jnp.float32)
         + jnp.dot(Tb[NP:NP + TROW], w1_ref[...],
                   preferred_element_type=jnp.float32)
         + jnp.dot(Tb[2 * NP:2 * NP + TROW], w2_ref[...],
                   preferred_element_type=jnp.float32) + bb)
    P = Y[:, 0:OUT]
    Q = Y[:, OUT:2 * OUT]
    R = Y[:, 2 * OUT:3 * OUT]
    H = jnp.maximum(P * (1.0 / (1.0 + jnp.exp(-Q))) + R, 0.0)
    Vf = jnp.dot(ST, H, preferred_element_type=jnp.float32)
    S2 = jnp.dot(ST, H * H, preferred_element_type=jnp.float32)
    h0 = H[0:NP]
    h1 = H[NP:2 * NP]
    h34 = H[(T2 - 2) * NP:(T2 - 1) * NP]
    h35 = H[(T2 - 1) * NP:T2 * NP]
    v_ref[0, 0] = Vf - h34 - h35
    v_ref[0, 1] = Vf - h0 - h35
    v_ref[0, 2] = Vf - h0 - h1
    s1 = jnp.sum(Vf, axis=1, keepdims=True)
    s2 = jnp.sum(S2, axis=1, keepdims=True)
    snew = jnp.concatenate([s1, s2], axis=1)
    b = pl.program_id(0)

    @pl.when(b == 0)
    def _():
        s_ref[...] = snew

    @pl.when(b > 0)
    def _():
        s_ref[...] = s_ref[...] + snew


def _final_body(v_ref, af_ref, df_ref, w3_ref, b3_ref, f1w_ref, f1b_ref,
                o_ref):
    af = af_ref[...]
    df = df_ref[...]
    acc = None
    for k in range(3):
        Vk = v_ref[:, k, :]
        U = Vk * af + float(T3) * df
        d = jnp.dot(U, w3_ref[k], preferred_element_type=jnp.float32)
        acc = d if acc is None else acc + d
    pooled = acc * (1.0 / float(T3)) + b3_ref[...]
    o_ref[...] = (jnp.dot(pooled, f1w_ref[...],
                          preferred_element_type=jnp.float32) + f1b_ref[...])


def kernel(X, edge_index, tc1_w1, tc1_b1, tc1_w2, tc1_b2, tc1_w3, tc1_b3,
           cheb_W, cheb_b, tc2_w1, tc2_b1, tc2_w2, tc2_b2, tc2_w3, tc2_b3,
           bn_gamma, bn_beta, conv3_w, conv3_b, f1_w, f1_b):
    f32 = jnp.float32
    ei = edge_index.reshape(NG, 2, E)
    A0 = _get_sc_kernel()(ei).reshape(NG, NP, NP)

    Xs = jnp.pad(X[..., 0], ((0, 0), (0, 0), (0, NP - N)))
    w1s = jnp.stack([tc1_w1[:, 0, 0, :].T, tc1_w2[:, 0, 0, :].T,
                     tc1_w3[:, 0, 0, :].T])
    b1s = jnp.stack([tc1_b1, tc1_b2, tc1_b3])

    def wcat(w):
        return jnp.transpose(w[:, :, 0, :], (2, 1, 0)).reshape(3 * HID, OUT)

    W2all = jnp.concatenate([wcat(tc2_w1), wcat(tc2_w2), wcat(tc2_w3)],
                            axis=1)
    b2all = jnp.concatenate([tc2_b1, tc2_b2, tc2_b3]).reshape(1, 3 * OUT)
    ii = jnp.arange(TROW, dtype=jnp.int32) % NP
    ST = (ii[None, :] == jnp.arange(NP, dtype=jnp.int32)[:, None]).astype(f32)
    V, S = pl.pallas_call(
        _cheb_tc2_body,
        grid=(B,),
        in_specs=[
            pl.BlockSpec((1, T1, NP, NP), lambda b: (b, 0, 0, 0)),
            pl.BlockSpec((1, T_IN, NP), lambda b: (b, 0, 0)),
            pl.BlockSpec((3, 3, HID), lambda b: (0, 0, 0)),
            pl.BlockSpec((3, HID), lambda b: (0, 0)),
            pl.BlockSpec((K, HID, HID), lambda b: (0, 0, 0)),
            pl.BlockSpec((1, HID), lambda b: (0, 0)),
            pl.BlockSpec((HID, 3 * OUT), lambda b: (0, 0)),
            pl.BlockSpec((HID, 3 * OUT), lambda b: (0, 0)),
            pl.BlockSpec((HID, 3 * OUT), lambda b: (0, 0)),
            pl.BlockSpec((1, 3 * OUT), lambda b: (0, 0)),
            pl.BlockSpec((NP, TROW), lambda b: (0, 0)),
        ],
        out_specs=[
            pl.BlockSpec((1, 3, NP, OUT), lambda b: (b, 0, 0, 0)),
            pl.BlockSpec((NP, 2), lambda b: (0, 0)),
        ],
        out_shape=[
            jax.ShapeDtypeStruct((B, 3, NP, OUT), f32),
            jax.ShapeDtypeStruct((NP, 2), f32),
        ],
    )(A0.reshape(B, T1, NP, NP), Xs, w1s, b1s, cheb_W,
      cheb_b.reshape(1, HID), W2all[0:HID], W2all[HID:2 * HID],
      W2all[2 * HID:3 * HID], b2all, ST)

    cnt = float(B * T2 * OUT)
    mu = S[:N, 0] / cnt
    var = S[:N, 1] / cnt - mu * mu
    sinv = lax.rsqrt(var + 1e-5)
    a = bn_gamma * sinv
    d = bn_beta - mu * a
    af = jnp.repeat(a, OUT).reshape(1, F)
    df = jnp.repeat(d, OUT).reshape(1, F)

    W3 = jnp.transpose(conv3_w[:, 0, :, :], (1, 2, 0))
    out = pl.pallas_call(
        _final_body,
        in_specs=[
            pl.BlockSpec((B, 3, F), lambda: (0, 0, 0)),
            pl.BlockSpec((1, F), lambda: (0, 0)),
            pl.BlockSpec((1, F), lambda: (0, 0)),
            pl.BlockSpec((3, F, 128), lambda: (0, 0, 0)),
            pl.BlockSpec((1, 128), lambda: (0, 0)),
            pl.BlockSpec((128, F), lambda: (0, 0)),
            pl.BlockSpec((1, F), lambda: (0, 0)),
        ],
        out_specs=pl.BlockSpec((B, F), lambda: (0, 0)),
        out_shape=jax.ShapeDtypeStruct((B, F), f32),
    )(V[:, :, :N, :].reshape(B, 3, F), af, df, W3, conv3_b.reshape(1, 128),
      f1_w, f1_b.reshape(1, F))
    return out


# speedup vs baseline: 1.1619x; 1.1619x over previous
import functools

import jax
import jax.numpy as jnp
from jax import lax
from jax.experimental import pallas as pl
from jax.experimental.pallas import tpu as pltpu
from jax.experimental.pallas import tpu_sc as plsc

B, T_IN, N, C_IN = 32, 40, 50, 1
HID, OUT, K, E = 32, 64, 3, 800
NG = B * (T_IN - 2)
T1 = T_IN - 2
T2 = T_IN - 4
T3 = T_IN - 6
NP = 56
APAD = NP * NP
GPW = NG // 32
F = N * OUT
TROW = T2 * NP


def _sc_body(ei_hbm, out_hbm, ei_a, ei_b, slot_a, slot_b, sem_in, sem_oa,
             sem_ob):
    cid = lax.axis_index("c")
    sid = lax.axis_index("s")
    wid = sid * 2 + cid
    g0 = wid * GPW
    zf = jnp.zeros((16,), jnp.float32)
    ones = jnp.full((16,), 1.0, jnp.float32)
    pltpu.async_copy(ei_hbm.at[g0], ei_a, sem_in)

    def one(g, ei_v, ei_nxt, slot_v, sem_out, first):
        pltpu.make_async_copy(ei_hbm.at[g], ei_v, sem_in).wait()

        @pl.when(g + 1 < g0 + GPW)
        def _():
            pltpu.async_copy(ei_hbm.at[g + 1], ei_nxt, sem_in)

        @pl.when(jnp.logical_not(first))
        def _():
            pltpu.make_async_copy(slot_v, out_hbm.at[g], sem_out).wait()
        for i in range(APAD // 16):
            slot_v[pl.ds(i * 16, 16)] = zf
        for i in range(E // 16):
            r = ei_v[0, pl.ds(i * 16, 16)]
            c = ei_v[1, pl.ds(i * 16, 16)]
            flat = r * NP + c
            ew = jnp.where(r != c, ones, zf)
            plsc.addupdate_scatter(slot_v, [flat], ew)
        pltpu.async_copy(slot_v, out_hbm.at[g], sem_out)

    def body(j, carry):
        g = g0 + 2 * j
        one(g, ei_a, ei_b, slot_a, sem_oa, j == 0)
        one(g + 1, ei_b, ei_a, slot_b, sem_ob, j == 0)
        return carry

    lax.fori_loop(0, GPW // 2, body, 0)
    pltpu.make_async_copy(slot_a, out_hbm.at[g0], sem_oa).wait()
    pltpu.make_async_copy(slot_b, out_hbm.at[g0], sem_ob).wait()


@functools.lru_cache(maxsize=1)
def _get_sc_kernel():
    mesh = plsc.VectorSubcoreMesh(core_axis_name="c", subcore_axis_name="s")
    return pl.kernel(
        _sc_body,
        mesh=mesh,
        compiler_params=pltpu.CompilerParams(needs_layout_passes=False),
        out_type=jax.ShapeDtypeStruct((NG, APAD), jnp.float32),
        scratch_types=[
            pltpu.VMEM((2, E), jnp.int32),
            pltpu.VMEM((2, E), jnp.int32),
            pltpu.VMEM((APAD,), jnp.float32),
            pltpu.VMEM((APAD,), jnp.float32),
            pltpu.SemaphoreType.DMA,
            pltpu.SemaphoreType.DMA,
            pltpu.SemaphoreType.DMA,
        ],
    )


def _tc1_body(x_ref, w_ref, b_ref, o_ref):
    x = x_ref[0]
    a = [x[k:k + T1][:, :, None] for k in range(3)]

    def conv(j):
        acc = b_ref[j][None, None, :]
        for k in range(3):
            acc = acc + a[k] * w_ref[j, k][None, None, :]
        return acc

    P = conv(0)
    Q = conv(1)
    R = conv(2)
    H = P * (1.0 / (1.0 + jnp.exp(-Q))) + R
    o_ref[0] = jnp.maximum(H, 0.0)


def _cheb_tc2_body(a_ref, t_ref, cw_ref, cb_ref, w0_ref,
                   b_ref, st_ref, v_ref, s_ref):
    W0 = cw_ref[0]
    W1 = cw_ref[1]
    W2 = cw_ref[2]
    cb = cb_ref[...]
    X_all = t_ref[0]
    A_all = a_ref[0].reshape(T1 * NP, NP)
    deg = jnp.sum(A_all, axis=1, keepdims=True)
    safe = jnp.where(deg > 0, deg, 1.0)
    dis = jnp.where(deg > 0, lax.rsqrt(safe), 0.0)
    ndis = -dis
    dx = dis * X_all
    t1 = [jnp.dot(a_ref[0, i], dx[i * NP:(i + 1) * NP],
                  preferred_element_type=jnp.float32) for i in range(T1)]
    Tx1 = ndis * jnp.concatenate(t1, axis=0)
    dt = dis * Tx1
    t2 = [jnp.dot(a_ref[0, i], dt[i * NP:(i + 1) * NP],
                  preferred_element_type=jnp.float32) for i in range(T1)]
    Tx2 = 2.0 * (ndis * jnp.concatenate(t2, axis=0)) - X_all
    Tg = (jnp.dot(X_all, W0, preferred_element_type=jnp.float32)
          + jnp.dot(Tx1, W1, preferred_element_type=jnp.float32)
          + jnp.dot(Tx2, W2, preferred_element_type=jnp.float32) + cb)
    Tb = jnp.maximum(Tg, 0.0)

    bb = b_ref[...]
    ST = st_ref[...]
    x3 = jnp.concatenate([Tb[0:TROW], Tb[NP:NP + TROW],
                          Tb[2 * NP:2 * NP + TROW]], axis=1)
    Y = jnp.dot(x3, w0_ref[...],
                preferred_element_type=jnp.float32) + bb
    P = Y[:, 0:OUT]
    Q = Y[:, OUT:2 * OUT]
    R = Y[:, 2 * OUT:3 * OUT]
    H = jnp.maximum(P * (1.0 / (1.0 + jnp.exp(-Q))) + R, 0.0)
    HH = jnp.concatenate([H, H * H], axis=1)
    VS = jnp.dot(ST, HH, preferred_element_type=jnp.float32)
    Vf = VS[:, 0:OUT]
    S2 = VS[:, OUT:2 * OUT]
    h0 = H[0:NP]
    h1 = H[NP:2 * NP]
    h34 = H[(T2 - 2) * NP:(T2 - 1) * NP]
    h35 = H[(T2 - 1) * NP:T2 * NP]
    v_ref[0, 0] = Vf - h34 - h35
    v_ref[0, 1] = Vf - h0 - h35
    v_ref[0, 2] = Vf - h0 - h1
    s1 = jnp.sum(Vf, axis=1, keepdims=True)
    s2 = jnp.sum(S2, axis=1, keepdims=True)
    snew = jnp.concatenate([s1, s2], axis=1)
    b = pl.program_id(0)

    @pl.when(b == 0)
    def _():
        s_ref[...] = snew

    @pl.when(b > 0)
    def _():
        s_ref[...] = s_ref[...] + snew


def _final_body(v_ref, af_ref, df_ref, w3_ref, b3_ref, f1w_ref, f1b_ref,
                o_ref):
    af = af_ref[...]
    df = df_ref[...]
    acc = None
    for k in range(3):
        Vk = v_ref[:, k, :]
        U = Vk * af + float(T3) * df
        d = jnp.dot(U, w3_ref[k], preferred_element_type=jnp.float32)
        acc = d if acc is None else acc + d
    pooled = acc * (1.0 / float(T3)) + b3_ref[...]
    o_ref[...] = (jnp.dot(pooled, f1w_ref[...],
                          preferred_element_type=jnp.float32) + f1b_ref[...])


def kernel(X, edge_index, tc1_w1, tc1_b1, tc1_w2, tc1_b2, tc1_w3, tc1_b3,
           cheb_W, cheb_b, tc2_w1, tc2_b1, tc2_w2, tc2_b2, tc2_w3, tc2_b3,
           bn_gamma, bn_beta, conv3_w, conv3_b, f1_w, f1_b):
    f32 = jnp.float32
    ei = edge_index.reshape(NG, 2, E)
    A0 = _get_sc_kernel()(ei).reshape(NG, NP, NP)

    Xs = jnp.pad(X[..., 0], ((0, 0), (0, 0), (0, NP - N)))
    w1s = jnp.stack([tc1_w1[:, 0, 0, :].T, tc1_w2[:, 0, 0, :].T,
                     tc1_w3[:, 0, 0, :].T])
    b1s = jnp.stack([tc1_b1, tc1_b2, tc1_b3])
    T0 = pl.pallas_call(
        _tc1_body,
        grid=(B,),
        in_specs=[
            pl.BlockSpec((1, T_IN, NP), lambda b: (b, 0, 0)),
            pl.BlockSpec((3, 3, HID), lambda b: (0, 0, 0)),
            pl.BlockSpec((3, HID), lambda b: (0, 0)),
        ],
        out_specs=pl.BlockSpec((1, T1, NP, HID), lambda b: (b, 0, 0, 0)),
        out_shape=jax.ShapeDtypeStruct((B, T1, NP, HID), f32),
    )(Xs, w1s, b1s)

    def wcat(w):
        return jnp.transpose(w[:, :, 0, :], (2, 1, 0)).reshape(3 * HID, OUT)

    W2all = jnp.concatenate([wcat(tc2_w1), wcat(tc2_w2), wcat(tc2_w3)],
                            axis=1)
    b2all = jnp.concatenate([tc2_b1, tc2_b2, tc2_b3]).reshape(1, 3 * OUT)
    ii = jnp.arange(TROW, dtype=jnp.int32) % NP
    ST = (ii[None, :] == jnp.arange(NP, dtype=jnp.int32)[:, None]).astype(f32)
    V, S = pl.pallas_call(
        _cheb_tc2_body,
        grid=(B,),
        in_specs=[
            pl.BlockSpec((1, T1, NP, NP), lambda b: (b, 0, 0, 0)),
            pl.BlockSpec((1, T1 * NP, HID), lambda b: (b, 0, 0)),
            pl.BlockSpec((K, HID, HID), lambda b: (0, 0, 0)),
            pl.BlockSpec((1, HID), lambda b: (0, 0)),
            pl.BlockSpec((3 * HID, 3 * OUT), lambda b: (0, 0)),
            pl.BlockSpec((1, 3 * OUT), lambda b: (0, 0)),
            pl.BlockSpec((NP, TROW), lambda b: (0, 0)),
        ],
        out_specs=[
            pl.BlockSpec((1, 3, NP, OUT), lambda b: (b, 0, 0, 0)),
            pl.BlockSpec((NP, 2), lambda b: (0, 0)),
        ],
        out_shape=[
            jax.ShapeDtypeStruct((B, 3, NP, OUT), f32),
            jax.ShapeDtypeStruct((NP, 2), f32),
        ],
    )(A0.reshape(B, T1, NP, NP), T0.reshape(B, T1 * NP, HID), cheb_W,
      cheb_b.reshape(1, HID), W2all, b2all, ST)

    cnt = float(B * T2 * OUT)
    mu = S[:N, 0] / cnt
    var = S[:N, 1] / cnt - mu * mu
    sinv = lax.rsqrt(var + 1e-5)
    a = bn_gamma * sinv
    d = bn_beta - mu * a
    af = jnp.repeat(a, OUT).reshape(1, F)
    df = jnp.repeat(d, OUT).reshape(1, F)

    W3 = jnp.transpose(conv3_w[:, 0, :, :], (1, 2, 0))
    out = pl.pallas_call(
        _final_body,
        in_specs=[
            pl.BlockSpec((B, 3, F), lambda: (0, 0, 0)),
            pl.BlockSpec((1, F), lambda: (0, 0)),
            pl.BlockSpec((1, F), lambda: (0, 0)),
            pl.BlockSpec((3, F, 128), lambda: (0, 0, 0)),
            pl.BlockSpec((1, 128), lambda: (0, 0)),
            pl.BlockSpec((128, F), lambda: (0, 0)),
            pl.BlockSpec((1, F), lambda: (0, 0)),
        ],
        out_specs=pl.BlockSpec((B, F), lambda: (0, 0)),
        out_shape=jax.ShapeDtypeStruct((B, F), f32),
    )(V[:, :, :N, :].reshape(B, 3, F), af, df, W3, conv3_b.reshape(1, 128),
      f1_w, f1_b.reshape(1, F))
    return out
